# baseline (speedup 1.0000x reference)
import functools

import jax
import jax.numpy as jnp
from jax import lax
from jax.experimental import pallas as pl
from jax.experimental.pallas import tpu as pltpu
from jax.experimental.pallas import tpu_sc as plsc

N_NODES = 10000
N_EDGES = 320000
D_FEAT = 128
TIME_WINDOW = 500

_B = 128
_K = 2
_TILES = 32
_NSLOT = 40
_EROWS = (_NSLOT + 1) * _TILES * _K
_NROWS = 10240
_ZROWS = _NROWS // 16


def _sc_body(x_hbm, src_hbm, dst_hbm, et_hbm, st_hbm, p_out, c_out,
             acc, accc, srcv, dstv, etv, stv, srcv1, dstv1, etv1, stv1,
             deff, rows, onesv, zb2, zb1,
             s_idx, s_st, s_idx1, s_st1, s_rows):
    cid = lax.axis_index("c")
    sid = lax.axis_index("s")
    wid = sid * 2 + cid

    z16 = jnp.zeros((16,), jnp.float32)
    for i in range(16):
        for j in range(8):
            zb2[i, pl.ds(j * 16, 16)] = z16
    for k in range(_ZROWS // 16):
        zb1[pl.ds(k * 16, 16)] = z16
    for j in range(8):
        onesv[pl.ds(j * 16, 16)] = jnp.ones((16,), jnp.float32)

    def zfire(k, carry):
        pltpu.make_async_copy(zb2, acc.at[pl.ds(sid * _ZROWS + k * 16, 16)],
                              s_idx).start()
        return carry

    def zwait(k, carry):
        pltpu.make_async_copy(zb2, acc.at[pl.ds(sid * _ZROWS + k * 16, 16)],
                              s_idx).wait()
        return carry

    lax.fori_loop(0, _ZROWS // 16, zfire, None)
    pltpu.sync_copy(zb1, accc.at[pl.ds(sid * _ZROWS, _ZROWS)])
    lax.fori_loop(0, _ZROWS // 16, zwait, None)

    plsc.subcore_barrier()

    iota16 = lax.iota(jnp.int32, 16)
    c240 = jnp.full((16,), 240, jnp.int32)

    def dummy_rows(k, j):
        rot = iota16 + jnp.full((16,), 16 * (8 * k + j), jnp.int32) + sid
        return N_NODES + lax.rem(rot, c240)

    bufs = [(srcv, dstv, etv, stv, s_idx, s_st),
            (srcv1, dstv1, etv1, stv1, s_idx1, s_st1)]

    def fire_idx(g, b):
        sv, dv, ev, _, si, _ = bufs[b]
        row0 = (g * _TILES + wid) * _K
        pltpu.make_async_copy(src_hbm.at[pl.ds(row0, _K)], sv, si).start()
        pltpu.make_async_copy(dst_hbm.at[pl.ds(row0, _K)], dv, si).start()
        pltpu.make_async_copy(et_hbm.at[pl.ds(row0, _K)], ev, si).start()

    def wait_idx(g, b):
        sv, dv, ev, _, si, _ = bufs[b]
        row0 = (g * _TILES + wid) * _K
        pltpu.make_async_copy(src_hbm.at[pl.ds(row0, _K)], sv, si).wait()
        pltpu.make_async_copy(dst_hbm.at[pl.ds(row0, _K)], dv, si).wait()
        pltpu.make_async_copy(et_hbm.at[pl.ds(row0, _K)], ev, si).wait()

    def fire_st(b):
        _, dv, _, tv, _, ss = bufs[b]
        for k in range(_K):
            pltpu.make_async_copy(st_hbm.at[dv.at[k]], tv.at[k], ss).start()

    def wait_st(b):
        _, dv, _, tv, _, ss = bufs[b]
        for k in range(_K):
            pltpu.make_async_copy(st_hbm.at[dv.at[k]], tv.at[k], ss).wait()

    def fire_rows(b, k):
        sv = bufs[b][0]
        pltpu.make_async_copy(x_hbm.at[sv.at[k]],
                              rows.at[pl.ds(k * _B, _B)], s_rows).start()

    def wait_rows(b, k):
        sv = bufs[b][0]
        pltpu.make_async_copy(x_hbm.at[sv.at[k]],
                              rows.at[pl.ds(k * _B, _B)], s_rows).wait()

    def do_slot(g, b):
        dv, ev, tv = bufs[b][1], bufs[b][2], bufs[b][3]
        fire_idx(g + 1, b ^ 1)
        wait_st(b)
        for k in range(_K):
            for j in range(_B // 16):
                sl = pl.ds(j * 16, 16)
                et = ev[k, sl]
                st = tv[k, sl]
                m = (et <= st) & (et > st - TIME_WINDOW)
                deff[k, sl] = jnp.where(m, dv[k, sl], dummy_rows(k, j))
        wait_idx(g + 1, b ^ 1)
        fire_st(b ^ 1)
        for k in range(_K):
            wait_rows(b, k)
            pltpu.sync_copy(rows.at[pl.ds(k * _B, _B)], acc.at[deff.at[k]],
                            add=True)
            pltpu.sync_copy(onesv, accc.at[deff.at[k]], add=True)
            fire_rows(b ^ 1, k)

    def pair(p, carry):
        do_slot(2 * p, 0)
        do_slot(2 * p + 1, 1)
        return carry

    fire_idx(0, 0)
    wait_idx(0, 0)
    fire_st(0)
    for k in range(_K):
        fire_rows(0, k)
    lax.fori_loop(0, _NSLOT // 2, pair, None)
    wait_st(0)
    for k in range(_K):
        wait_rows(0, k)

    plsc.subcore_barrier()

    pltpu.sync_copy(acc.at[pl.ds(sid * _ZROWS, _ZROWS)],
                    p_out.at[pl.ds(cid * _NROWS + sid * _ZROWS, _ZROWS)])
    pltpu.sync_copy(accc.at[pl.ds(sid * _ZROWS, _ZROWS)],
                    c_out.at[pl.ds(cid * _NROWS + sid * _ZROWS, _ZROWS)])


_sc_call = functools.partial(
    pl.kernel,
    out_type=[
        jax.ShapeDtypeStruct((2 * _NROWS, D_FEAT), jnp.float32),
        jax.ShapeDtypeStruct((2 * _NROWS,), jnp.float32),
    ],
    mesh=plsc.VectorSubcoreMesh(core_axis_name="c", subcore_axis_name="s"),
    scratch_types=[
        pltpu.VMEM_SHARED((_NROWS, D_FEAT), jnp.float32),
        pltpu.VMEM_SHARED((_NROWS,), jnp.float32),
        pltpu.VMEM((_K, _B), jnp.int32),
        pltpu.VMEM((_K, _B), jnp.int32),
        pltpu.VMEM((_K, _B), jnp.int32),
        pltpu.VMEM((_K, _B), jnp.int32),
        pltpu.VMEM((_K, _B), jnp.int32),
        pltpu.VMEM((_K, _B), jnp.int32),
        pltpu.VMEM((_K, _B), jnp.int32),
        pltpu.VMEM((_K, _B), jnp.int32),
        pltpu.VMEM((_K, _B), jnp.int32),
        pltpu.VMEM((_K * _B, D_FEAT), jnp.float32),
        pltpu.VMEM((_B,), jnp.float32),
        pltpu.VMEM((16, D_FEAT), jnp.float32),
        pltpu.VMEM((_ZROWS,), jnp.float32),
        pltpu.SemaphoreType.DMA,
        pltpu.SemaphoreType.DMA,
        pltpu.SemaphoreType.DMA,
        pltpu.SemaphoreType.DMA,
        pltpu.SemaphoreType.DMA,
    ],
)(_sc_body)


def _combine_body(x_ref, p0_ref, p1_ref, c0_ref, c1_ref, o_ref):
    cnt = c0_ref[0, 0, :] + c1_ref[0, 0, :]
    s = p0_ref[...] + p1_ref[...]
    o_ref[...] = x_ref[...] + s / jnp.clip(cnt, 1.0, None)[:, None]


_R = 1000


def _combine(x, p0, p1, c0, c1):
    return pl.pallas_call(
        _combine_body,
        grid=(N_NODES // _R,),
        in_specs=[
            pl.BlockSpec((_R, D_FEAT), lambda i: (i, 0)),
            pl.BlockSpec((_R, D_FEAT), lambda i: (i, 0)),
            pl.BlockSpec((_R, D_FEAT), lambda i: (i, 0)),
            pl.BlockSpec((1, 1, _R), lambda i: (i, 0, 0)),
            pl.BlockSpec((1, 1, _R), lambda i: (i, 0, 0)),
        ],
        out_specs=pl.BlockSpec((_R, D_FEAT), lambda i: (i, 0)),
        out_shape=jax.ShapeDtypeStruct((N_NODES, D_FEAT), jnp.float32),
    )(x, p0, p1, c0, c1)


@jax.jit
def kernel(x, edge_index, edge_time, seed_time):
    pad = _EROWS * _B - N_EDGES
    spread = jnp.arange(pad, dtype=jnp.int32) % N_NODES
    src = jnp.concatenate([edge_index[0], spread]).reshape(_EROWS, _B)
    dst = jnp.concatenate([edge_index[1], spread]).reshape(_EROWS, _B)
    et = jnp.concatenate(
        [edge_time, jnp.full((pad,), 2 ** 30, jnp.int32)]).reshape(_EROWS, _B)
    pr, cr = _sc_call(x, src, dst, et, seed_time)
    p0 = pr[:N_NODES]
    p1 = pr[_NROWS:_NROWS + N_NODES]
    c0 = cr[:N_NODES].reshape(N_NODES // _R, 1, _R)
    c1 = cr[_NROWS:_NROWS + N_NODES].reshape(N_NODES // _R, 1, _R)
    return _combine(x, p0, p1, c0, c1)


# speedup vs baseline: 1.0047x; 1.0047x over previous
import functools

import jax
import jax.numpy as jnp
from jax import lax
from jax.experimental import pallas as pl
from jax.experimental.pallas import tpu as pltpu
from jax.experimental.pallas import tpu_sc as plsc

N_NODES = 10000
N_EDGES = 320000
D_FEAT = 128
TIME_WINDOW = 500

_B = 128
_K = 2
_TILES = 32
_NSLOT = 40
_EROWS = (_NSLOT + 1) * _TILES * _K
_NROWS = 10240
_ZROWS = _NROWS // 16


def _sc_body(x_hbm, src_hbm, dst_hbm, et_hbm, st_hbm, p_out, c_out,
             acc, accc, srcv, dstv, etv, stv, srcv1, dstv1, etv1, stv1,
             deff, rows, onesv, zb2, zb1,
             s_idx, s_st, s_idx1, s_st1, s_rows):
    cid = lax.axis_index("c")
    sid = lax.axis_index("s")
    wid = sid * 2 + cid

    z16 = jnp.zeros((16,), jnp.float32)
    for i in range(16):
        for j in range(8):
            zb2[i, pl.ds(j * 16, 16)] = z16
    for k in range(_ZROWS // 16):
        zb1[pl.ds(k * 16, 16)] = z16
    for j in range(8):
        onesv[pl.ds(j * 16, 16)] = jnp.ones((16,), jnp.float32)

    def zfire(k, carry):
        pltpu.make_async_copy(zb2, acc.at[pl.ds(sid * _ZROWS + k * 16, 16)],
                              s_idx).start()
        return carry

    def zwait(k, carry):
        pltpu.make_async_copy(zb2, acc.at[pl.ds(sid * _ZROWS + k * 16, 16)],
                              s_idx).wait()
        return carry

    lax.fori_loop(0, _ZROWS // 16, zfire, None)
    pltpu.sync_copy(zb1, accc.at[pl.ds(sid * _ZROWS, _ZROWS)])
    lax.fori_loop(0, _ZROWS // 16, zwait, None)

    plsc.subcore_barrier()

    iota16 = lax.iota(jnp.int32, 16)
    c240 = jnp.full((16,), 240, jnp.int32)

    def dummy_rows(k, j):
        rot = iota16 + jnp.full((16,), 16 * (8 * k + j), jnp.int32) + sid
        return N_NODES + lax.rem(rot, c240)

    bufs = [(srcv, dstv, etv, stv, s_idx, s_st),
            (srcv1, dstv1, etv1, stv1, s_idx1, s_st1)]

    def fire_idx(g, b):
        sv, dv, ev, _, si, _ = bufs[b]
        row0 = (g * _TILES + wid) * _K
        pltpu.make_async_copy(src_hbm.at[pl.ds(row0, _K)], sv, si).start()
        pltpu.make_async_copy(dst_hbm.at[pl.ds(row0, _K)], dv, si).start()
        pltpu.make_async_copy(et_hbm.at[pl.ds(row0, _K)], ev, si).start()

    def wait_idx(g, b):
        sv, dv, ev, _, si, _ = bufs[b]
        row0 = (g * _TILES + wid) * _K
        pltpu.make_async_copy(src_hbm.at[pl.ds(row0, _K)], sv, si).wait()
        pltpu.make_async_copy(dst_hbm.at[pl.ds(row0, _K)], dv, si).wait()
        pltpu.make_async_copy(et_hbm.at[pl.ds(row0, _K)], ev, si).wait()

    def fire_st(b):
        _, dv, _, tv, _, ss = bufs[b]
        for k in range(_K):
            pltpu.make_async_copy(st_hbm.at[dv.at[k]], tv.at[k], ss).start()

    def wait_st(b):
        _, dv, _, tv, _, ss = bufs[b]
        for k in range(_K):
            pltpu.make_async_copy(st_hbm.at[dv.at[k]], tv.at[k], ss).wait()

    def fire_rows(b, k):
        sv = bufs[b][0]
        pltpu.make_async_copy(x_hbm.at[sv.at[k]],
                              rows.at[pl.ds(k * _B, _B)], s_rows).start()

    def wait_rows(b, k):
        sv = bufs[b][0]
        pltpu.make_async_copy(x_hbm.at[sv.at[k]],
                              rows.at[pl.ds(k * _B, _B)], s_rows).wait()

    def do_slot(g, b):
        dv, ev, tv = bufs[b][1], bufs[b][2], bufs[b][3]
        fire_idx(g + 1, b ^ 1)
        wait_st(b)
        for k in range(_K):
            for j in range(_B // 16):
                sl = pl.ds(j * 16, 16)
                et = ev[k, sl]
                st = tv[k, sl]
                m = (et <= st) & (et > st - TIME_WINDOW)
                deff[k, sl] = jnp.where(m, dv[k, sl], dummy_rows(k, j))
        for k in range(_K):
            pltpu.sync_copy(onesv, accc.at[deff.at[k]], add=True)
        wait_idx(g + 1, b ^ 1)
        fire_st(b ^ 1)
        for k in range(_K):
            wait_rows(b, k)
            pltpu.sync_copy(rows.at[pl.ds(k * _B, _B)], acc.at[deff.at[k]],
                            add=True)
            fire_rows(b ^ 1, k)

    def pair(p, carry):
        do_slot(2 * p, 0)
        do_slot(2 * p + 1, 1)
        return carry

    fire_idx(0, 0)
    wait_idx(0, 0)
    fire_st(0)
    for k in range(_K):
        fire_rows(0, k)
    lax.fori_loop(0, _NSLOT // 2, pair, None)
    wait_st(0)
    for k in range(_K):
        wait_rows(0, k)

    plsc.subcore_barrier()

    pltpu.sync_copy(acc.at[pl.ds(sid * _ZROWS, _ZROWS)],
                    p_out.at[pl.ds(cid * _NROWS + sid * _ZROWS, _ZROWS)])
    pltpu.sync_copy(accc.at[pl.ds(sid * _ZROWS, _ZROWS)],
                    c_out.at[pl.ds(cid * _NROWS + sid * _ZROWS, _ZROWS)])


_sc_call = functools.partial(
    pl.kernel,
    out_type=[
        jax.ShapeDtypeStruct((2 * _NROWS, D_FEAT), jnp.float32),
        jax.ShapeDtypeStruct((2 * _NROWS,), jnp.float32),
    ],
    mesh=plsc.VectorSubcoreMesh(core_axis_name="c", subcore_axis_name="s"),
    scratch_types=[
        pltpu.VMEM_SHARED((_NROWS, D_FEAT), jnp.float32),
        pltpu.VMEM_SHARED((_NROWS,), jnp.float32),
        pltpu.VMEM((_K, _B), jnp.int32),
        pltpu.VMEM((_K, _B), jnp.int32),
        pltpu.VMEM((_K, _B), jnp.int32),
        pltpu.VMEM((_K, _B), jnp.int32),
        pltpu.VMEM((_K, _B), jnp.int32),
        pltpu.VMEM((_K, _B), jnp.int32),
        pltpu.VMEM((_K, _B), jnp.int32),
        pltpu.VMEM((_K, _B), jnp.int32),
        pltpu.VMEM((_K, _B), jnp.int32),
        pltpu.VMEM((_K * _B, D_FEAT), jnp.float32),
        pltpu.VMEM((_B,), jnp.float32),
        pltpu.VMEM((16, D_FEAT), jnp.float32),
        pltpu.VMEM((_ZROWS,), jnp.float32),
        pltpu.SemaphoreType.DMA,
        pltpu.SemaphoreType.DMA,
        pltpu.SemaphoreType.DMA,
        pltpu.SemaphoreType.DMA,
        pltpu.SemaphoreType.DMA,
    ],
)(_sc_body)


def _combine_body(x_ref, p0_ref, p1_ref, c0_ref, c1_ref, o_ref):
    cnt = c0_ref[0, 0, :] + c1_ref[0, 0, :]
    s = p0_ref[...] + p1_ref[...]
    o_ref[...] = x_ref[...] + s / jnp.clip(cnt, 1.0, None)[:, None]


_R = 1000


def _combine(x, p0, p1, c0, c1):
    return pl.pallas_call(
        _combine_body,
        grid=(N_NODES // _R,),
        in_specs=[
            pl.BlockSpec((_R, D_FEAT), lambda i: (i, 0)),
            pl.BlockSpec((_R, D_FEAT), lambda i: (i, 0)),
            pl.BlockSpec((_R, D_FEAT), lambda i: (i, 0)),
            pl.BlockSpec((1, 1, _R), lambda i: (i, 0, 0)),
            pl.BlockSpec((1, 1, _R), lambda i: (i, 0, 0)),
        ],
        out_specs=pl.BlockSpec((_R, D_FEAT), lambda i: (i, 0)),
        out_shape=jax.ShapeDtypeStruct((N_NODES, D_FEAT), jnp.float32),
    )(x, p0, p1, c0, c1)


@jax.jit
def kernel(x, edge_index, edge_time, seed_time):
    pad = _EROWS * _B - N_EDGES
    spread = jnp.arange(pad, dtype=jnp.int32) % N_NODES
    src = jnp.concatenate([edge_index[0], spread]).reshape(_EROWS, _B)
    dst = jnp.concatenate([edge_index[1], spread]).reshape(_EROWS, _B)
    et = jnp.concatenate(
        [edge_time, jnp.full((pad,), 2 ** 30, jnp.int32)]).reshape(_EROWS, _B)
    pr, cr = _sc_call(x, src, dst, et, seed_time)
    p0 = pr[:N_NODES]
    p1 = pr[_NROWS:_NROWS + N_NODES]
    c0 = cr[:N_NODES].reshape(N_NODES // _R, 1, _R)
    c1 = cr[_NROWS:_NROWS + N_NODES].reshape(N_NODES // _R, 1, _R)
    return _combine(x, p0, p1, c0, c1)
